# baseline (speedup 1.0000x reference)
import functools
import jax
import jax.numpy as jnp
from jax import lax
from jax.experimental import pallas as pl
from jax.experimental.pallas import tpu as pltpu
from jax.experimental.pallas import tpu_sc as plsc

N_U = 25000
N_I = 25000
N = N_U + N_I
E = 800000
D = 64
H = D // 2

NUM_CORES = 2
NUM_TILES = 16
BLK = 256
CHUNK = 128
CPB = BLK // CHUNK
EPT_BLKS = 196
EPT = EPT_BLKS * BLK
E_PAD = NUM_TILES * EPT
RPT = 3128
RPT_LAST = N - 15 * RPT


def _spmm_body(ego_hbm, rows_hbm, cols_hbm, vals_hbm, zeros_hbm, out_hbm,
               rowsb, gidxb, colsb, valsb, gbuf, accum,
               isem, gsem, ssem):
  c = lax.axis_index("c")
  t = lax.axis_index("s")
  NB = EPT_BLKS

  def idx_fire(b, p):
    base = pl.multiple_of((t * NB + b) * BLK, BLK)
    roff = pl.multiple_of((t * NB + b) * CPB, CPB)
    pltpu.async_copy(cols_hbm.at[pl.ds(base, BLK)], colsb.at[p], isem)
    pltpu.async_copy(vals_hbm.at[pl.ds(base, BLK)], valsb.at[p], isem)
    pltpu.async_copy(rows_hbm.at[pl.ds(roff, CPB)], rowsb.at[p], isem)

  def idx_drain(p):
    pltpu.make_async_copy(cols_hbm.at[pl.ds(0, BLK)], colsb.at[p], isem).wait()
    pltpu.make_async_copy(vals_hbm.at[pl.ds(0, BLK)], valsb.at[p], isem).wait()
    pltpu.make_async_copy(rows_hbm.at[pl.ds(0, CPB)], rowsb.at[p], isem).wait()

  def gidx_compute(p):
    def row(k, carry):
      for q in range(CHUNK // 16):
        g = colsb[p, pl.ds(k * CHUNK + q * 16, 16)]
        gidxb[p, k, pl.ds(q * 16, 16)] = g + g + c
      return carry
    lax.fori_loop(0, CPB, row, 0)

  def gather_fire(p, g):
    for j in range(CPB):
      pltpu.async_copy(ego_hbm.at[gidxb.at[p, j]],
                       gbuf.at[g, pl.ds(j * CHUNK, CHUNK)], gsem)

  def gather_drain(p, g):
    for j in range(CPB):
      pltpu.make_async_copy(ego_hbm.at[gidxb.at[p, j]],
                            gbuf.at[g, pl.ds(j * CHUNK, CHUNK)], gsem).wait()

  def scale(p, g):
    def group(i, carry):
      v = valsb[p, pl.ds(i * 16, 16)]
      for e in range(16):
        valv = jnp.broadcast_to(v[e], (16,))
        r = i * 16 + e
        gbuf[g, r, pl.ds(0, 16)] = gbuf[g, r, pl.ds(0, 16)] * valv
        gbuf[g, r, pl.ds(16, 16)] = gbuf[g, r, pl.ds(16, 16)] * valv
      return carry
    lax.fori_loop(0, BLK // 16, group, 0)

  def scatter_fire(p, g):
    for j in range(CPB):
      pltpu.async_copy(gbuf.at[g, pl.ds(j * CHUNK, CHUNK)],
                       accum.at[rowsb.at[p, j]], ssem, add=True)

  def scatter_drain(p, g):
    for j in range(CPB):
      pltpu.make_async_copy(gbuf.at[g, pl.ds(j * CHUNK, CHUNK)],
                            accum.at[rowsb.at[p, j]], ssem).wait()

  off = pl.multiple_of(t * RPT, 8)

  @pl.when(t < NUM_TILES - 1)
  def _():
    pltpu.sync_copy(zeros_hbm, accum.at[pl.ds(off, RPT)])

  @pl.when(t == NUM_TILES - 1)
  def _():
    pltpu.sync_copy(zeros_hbm.at[pl.ds(0, RPT_LAST)],
                    accum.at[pl.ds(off, RPT_LAST)])

  plsc.subcore_barrier()

  idx_fire(0, 0)
  idx_drain(0)
  gidx_compute(0)
  gather_fire(0, 0)
  idx_fire(1, 1)

  def block(b, carry):
    p4 = lax.rem(b, 4)
    g3 = lax.rem(b, 3)
    q4 = lax.rem(b + 1, 4)
    h3 = lax.rem(b + 1, 3)

    gather_drain(p4, g3)

    @pl.when(b + 1 < NB)
    def _():
      idx_drain(q4)
      gidx_compute(q4)
      gather_fire(q4, h3)

      @pl.when(b + 2 < NB)
      def _():
        idx_fire(b + 2, lax.rem(b + 2, 4))

    scale(p4, g3)

    @pl.when(b >= 1)
    def _():
      scatter_drain(lax.rem(b + 3, 4), lax.rem(b + 2, 3))

    scatter_fire(p4, g3)
    return carry

  lax.fori_loop(0, NB, block, 0)
  scatter_drain(lax.rem(NB - 1, 4), lax.rem(NB - 1, 3))
  plsc.subcore_barrier()

  @pl.when(t < NUM_TILES - 1)
  def _():
    sl = pl.ds(off, RPT)
    pltpu.sync_copy(accum.at[sl], out_hbm.at[c, sl])

  @pl.when(t == NUM_TILES - 1)
  def _():
    sl = pl.ds(off, RPT_LAST)
    pltpu.sync_copy(accum.at[sl], out_hbm.at[c, sl])


_spmm = pl.kernel(
    _spmm_body,
    out_type=jax.ShapeDtypeStruct((NUM_CORES, N, H), jnp.float32),
    mesh=plsc.VectorSubcoreMesh(core_axis_name="c", subcore_axis_name="s"),
    scratch_types=[
        pltpu.VMEM((4, CPB, CHUNK), jnp.int32),
        pltpu.VMEM((4, CPB, CHUNK), jnp.int32),
        pltpu.VMEM((4, BLK), jnp.int32),
        pltpu.VMEM((4, BLK), jnp.float32),
        pltpu.VMEM((3, BLK, H), jnp.float32),
        pltpu.VMEM_SHARED((N, H), jnp.float32),
        pltpu.SemaphoreType.DMA,
        pltpu.SemaphoreType.DMA,
        pltpu.SemaphoreType.DMA,
    ],
    compiler_params=pltpu.CompilerParams(use_tc_tiling_on_sc=False),
)


def _dense_body(ego_ref, h0_ref, h1_ref, wg_ref, bg_ref, wb_ref, bb_ref,
                enext_ref, norm_ref):
  s = jnp.concatenate([h0_ref[...], h1_ref[...]], axis=1)
  ego = ego_ref[...]
  x = jnp.dot(s, wg_ref[...], preferred_element_type=jnp.float32) + bg_ref[...]
  sum_emb = jnp.where(x > 0, x, 0.01 * x)
  y = jnp.dot(ego * s, wb_ref[...], preferred_element_type=jnp.float32) + bb_ref[...]
  bi = jnp.where(y > 0, y, 0.01 * y)
  e2 = sum_emb + bi
  nrm = jnp.sqrt(jnp.sum(e2 * e2, axis=1, keepdims=True))
  enext_ref[...] = e2
  norm_ref[...] = e2 / jnp.maximum(nrm, 1e-12)


_BN = 2000


def _dense(ego, h0, h1, wgt, bg, wbt, bb):
  return pl.pallas_call(
      _dense_body,
      grid=(N // _BN,),
      in_specs=[
          pl.BlockSpec((_BN, D), lambda i: (i, 0)),
          pl.BlockSpec((_BN, H), lambda i: (i, 0)),
          pl.BlockSpec((_BN, H), lambda i: (i, 0)),
          pl.BlockSpec((D, D), lambda i: (0, 0)),
          pl.BlockSpec((1, D), lambda i: (0, 0)),
          pl.BlockSpec((D, D), lambda i: (0, 0)),
          pl.BlockSpec((1, D), lambda i: (0, 0)),
      ],
      out_specs=[
          pl.BlockSpec((_BN, D), lambda i: (i, 0)),
          pl.BlockSpec((_BN, D), lambda i: (i, 0)),
      ],
      out_shape=[
          jax.ShapeDtypeStruct((N, D), jnp.float32),
          jax.ShapeDtypeStruct((N, D), jnp.float32),
      ],
  )(ego, h0, h1, wgt, bg, wbt, bb)


def kernel(adj_rows, adj_cols, adj_vals, user_emb, item_emb,
           W_gc0, b_gc0, W_bi0, b_bi0, W_gc1, b_gc1, W_bi1, b_bi1):
  rows = adj_rows.astype(jnp.int32)
  cols = adj_cols.astype(jnp.int32)
  vals = adj_vals.astype(jnp.float32)
  pad = E_PAD - E
  rows_p = jnp.concatenate([rows, jnp.zeros((pad,), jnp.int32)])
  cols_p = jnp.concatenate([cols, jnp.zeros((pad,), jnp.int32)])
  vals_p = jnp.concatenate([vals, jnp.zeros((pad,), jnp.float32)])
  rows2d = rows_p.reshape(E_PAD // CHUNK, CHUNK)
  zeros = jnp.zeros((RPT, H), jnp.float32)

  ego0 = jnp.concatenate([user_emb, item_emb], axis=0)
  params = [
      (W_gc0.T, b_gc0.reshape(1, D), W_bi0.T, b_bi0.reshape(1, D)),
      (W_gc1.T, b_gc1.reshape(1, D), W_bi1.T, b_bi1.reshape(1, D)),
  ]

  ego = ego0
  norms = []
  for (wgt, bg, wbt, bb) in params:
    side = _spmm(ego.reshape(2 * N, H), rows2d, cols_p, vals_p, zeros)
    ego, norm = _dense(ego, side[0], side[1], wgt, bg, wbt, bb)
    norms.append(norm)

  all_emb = jnp.concatenate([ego0, norms[0], norms[1]], axis=1)
  return all_emb[:N_U], all_emb[N_U:]


# speedup vs baseline: 1.5488x; 1.5488x over previous
import functools
import jax
import jax.numpy as jnp
from jax import lax
from jax.experimental import pallas as pl
from jax.experimental.pallas import tpu as pltpu
from jax.experimental.pallas import tpu_sc as plsc

N_U = 25000
N_I = 25000
N = N_U + N_I
E = 800000
D = 64
H = D // 2

NUM_CORES = 2
NUM_TILES = 16
BLK = 256
CHUNK = 128
CPB = BLK // CHUNK
EPT_BLKS = 196
EPT = EPT_BLKS * BLK
E_PAD = NUM_TILES * EPT
RPT = 3128
RPT_LAST = N - 15 * RPT


def _spmm_body(ego_hbm, rows_hbm, cols_hbm, vals_hbm, zeros_hbm, out_hbm,
               rowsb, gidxb, colsb, valsb, gbuf, accum,
               isem, gsem, ssem):
  c = lax.axis_index("c")
  t = lax.axis_index("s")
  NB = EPT_BLKS

  def idx_fire(b, p):
    base = pl.multiple_of((t * NB + b) * BLK, BLK)
    roff = pl.multiple_of((t * NB + b) * CPB, CPB)
    pltpu.async_copy(cols_hbm.at[pl.ds(base, BLK)], colsb.at[p], isem)
    pltpu.async_copy(vals_hbm.at[pl.ds(base, BLK)], valsb.at[p], isem)
    pltpu.async_copy(rows_hbm.at[pl.ds(roff, CPB)], rowsb.at[p], isem)

  def idx_drain(p):
    pltpu.make_async_copy(cols_hbm.at[pl.ds(0, BLK)], colsb.at[p], isem).wait()
    pltpu.make_async_copy(vals_hbm.at[pl.ds(0, BLK)], valsb.at[p], isem).wait()
    pltpu.make_async_copy(rows_hbm.at[pl.ds(0, CPB)], rowsb.at[p], isem).wait()

  def gidx_compute(p):
    def row(k, carry):
      for q in range(CHUNK // 16):
        g = colsb[p, pl.ds(k * CHUNK + q * 16, 16)]
        gidxb[p, k, pl.ds(q * 16, 16)] = g + g + c
      return carry
    lax.fori_loop(0, CPB, row, 0)

  def gather_fire(p, g):
    for j in range(CPB):
      pltpu.async_copy(ego_hbm.at[gidxb.at[p, j]],
                       gbuf.at[g, pl.ds(j * CHUNK, CHUNK)], gsem)

  def gather_drain(p, g):
    for j in range(CPB):
      pltpu.make_async_copy(ego_hbm.at[gidxb.at[p, j]],
                            gbuf.at[g, pl.ds(j * CHUNK, CHUNK)], gsem).wait()

  def scale(p, g):
    def group(i, carry):
      v = valsb[p, pl.ds(i * 16, 16)]
      for e in range(16):
        valv = jnp.broadcast_to(v[e], (16,))
        r = i * 16 + e
        gbuf[g, r, pl.ds(0, 16)] = gbuf[g, r, pl.ds(0, 16)] * valv
        gbuf[g, r, pl.ds(16, 16)] = gbuf[g, r, pl.ds(16, 16)] * valv
      return carry
    lax.fori_loop(0, BLK // 16, group, 0)

  def scatter_fire(p, g):
    for j in range(CPB):
      pltpu.async_copy(gbuf.at[g, pl.ds(j * CHUNK, CHUNK)],
                       accum.at[rowsb.at[p, j]], ssem, add=True)

  def scatter_drain(p, g):
    for j in range(CPB):
      pltpu.make_async_copy(gbuf.at[g, pl.ds(j * CHUNK, CHUNK)],
                            accum.at[rowsb.at[p, j]], ssem).wait()

  off = pl.multiple_of(t * RPT, 8)

  @pl.when(t < NUM_TILES - 1)
  def _():
    pltpu.sync_copy(zeros_hbm, accum.at[pl.ds(off, RPT)])

  @pl.when(t == NUM_TILES - 1)
  def _():
    pltpu.sync_copy(zeros_hbm.at[pl.ds(0, RPT_LAST)],
                    accum.at[pl.ds(off, RPT_LAST)])

  plsc.subcore_barrier()

  idx_fire(0, 0)
  idx_drain(0)
  gidx_compute(0)
  gather_fire(0, 0)
  idx_fire(1, 1)

  def bump(x, m):
    return jnp.where(x == m - 1, 0, x + 1)

  def block(b, carry):
    (p4, g3, q4, h3, r4, s4, u3) = carry

    gather_drain(p4, g3)

    @pl.when(b + 1 < NB)
    def _():
      idx_drain(q4)
      gidx_compute(q4)
      gather_fire(q4, h3)

      @pl.when(b + 2 < NB)
      def _():
        idx_fire(b + 2, r4)

    scale(p4, g3)

    @pl.when(b >= 1)
    def _():
      scatter_drain(s4, u3)

    scatter_fire(p4, g3)
    return (q4, h3, r4, bump(h3, 3), bump(r4, 4), p4, g3)

  init = (jnp.int32(0), jnp.int32(0), jnp.int32(1), jnp.int32(1),
          jnp.int32(2), jnp.int32(3), jnp.int32(2))
  last = lax.fori_loop(0, NB, block, init)
  scatter_drain(last[5], last[6])
  plsc.subcore_barrier()

  @pl.when(t < NUM_TILES - 1)
  def _():
    sl = pl.ds(off, RPT)
    pltpu.sync_copy(accum.at[sl], out_hbm.at[c, sl])

  @pl.when(t == NUM_TILES - 1)
  def _():
    sl = pl.ds(off, RPT_LAST)
    pltpu.sync_copy(accum.at[sl], out_hbm.at[c, sl])


_spmm = pl.kernel(
    _spmm_body,
    out_type=jax.ShapeDtypeStruct((NUM_CORES, N, H), jnp.float32),
    mesh=plsc.VectorSubcoreMesh(core_axis_name="c", subcore_axis_name="s"),
    scratch_types=[
        pltpu.VMEM((4, CPB, CHUNK), jnp.int32),
        pltpu.VMEM((4, CPB, CHUNK), jnp.int32),
        pltpu.VMEM((4, BLK), jnp.int32),
        pltpu.VMEM((4, BLK), jnp.float32),
        pltpu.VMEM((3, BLK, H), jnp.float32),
        pltpu.VMEM_SHARED((N, H), jnp.float32),
        pltpu.SemaphoreType.DMA,
        pltpu.SemaphoreType.DMA,
        pltpu.SemaphoreType.DMA,
    ],
    compiler_params=pltpu.CompilerParams(use_tc_tiling_on_sc=False),
)


def _dense_body(ego_ref, h0_ref, h1_ref, wg_ref, bg_ref, wb_ref, bb_ref,
                enext_ref, norm_ref):
  s = jnp.concatenate([h0_ref[...], h1_ref[...]], axis=1)
  ego = ego_ref[...]
  x = jnp.dot(s, wg_ref[...], preferred_element_type=jnp.float32) + bg_ref[...]
  sum_emb = jnp.where(x > 0, x, 0.01 * x)
  y = jnp.dot(ego * s, wb_ref[...], preferred_element_type=jnp.float32) + bb_ref[...]
  bi = jnp.where(y > 0, y, 0.01 * y)
  e2 = sum_emb + bi
  nrm = jnp.sqrt(jnp.sum(e2 * e2, axis=1, keepdims=True))
  enext_ref[...] = e2
  norm_ref[...] = e2 / jnp.maximum(nrm, 1e-12)


_BN = 2000


def _dense(ego, h0, h1, wgt, bg, wbt, bb):
  return pl.pallas_call(
      _dense_body,
      grid=(N // _BN,),
      in_specs=[
          pl.BlockSpec((_BN, D), lambda i: (i, 0)),
          pl.BlockSpec((_BN, H), lambda i: (i, 0)),
          pl.BlockSpec((_BN, H), lambda i: (i, 0)),
          pl.BlockSpec((D, D), lambda i: (0, 0)),
          pl.BlockSpec((1, D), lambda i: (0, 0)),
          pl.BlockSpec((D, D), lambda i: (0, 0)),
          pl.BlockSpec((1, D), lambda i: (0, 0)),
      ],
      out_specs=[
          pl.BlockSpec((_BN, D), lambda i: (i, 0)),
          pl.BlockSpec((_BN, D), lambda i: (i, 0)),
      ],
      out_shape=[
          jax.ShapeDtypeStruct((N, D), jnp.float32),
          jax.ShapeDtypeStruct((N, D), jnp.float32),
      ],
  )(ego, h0, h1, wgt, bg, wbt, bb)


def kernel(adj_rows, adj_cols, adj_vals, user_emb, item_emb,
           W_gc0, b_gc0, W_bi0, b_bi0, W_gc1, b_gc1, W_bi1, b_bi1):
  rows = adj_rows.astype(jnp.int32)
  cols = adj_cols.astype(jnp.int32)
  vals = adj_vals.astype(jnp.float32)
  pad = E_PAD - E
  rows_p = jnp.concatenate([rows, jnp.zeros((pad,), jnp.int32)])
  cols_p = jnp.concatenate([cols, jnp.zeros((pad,), jnp.int32)])
  vals_p = jnp.concatenate([vals, jnp.zeros((pad,), jnp.float32)])
  rows2d = rows_p.reshape(E_PAD // CHUNK, CHUNK)
  zeros = jnp.zeros((RPT, H), jnp.float32)

  ego0 = jnp.concatenate([user_emb, item_emb], axis=0)
  params = [
      (W_gc0.T, b_gc0.reshape(1, D), W_bi0.T, b_bi0.reshape(1, D)),
      (W_gc1.T, b_gc1.reshape(1, D), W_bi1.T, b_bi1.reshape(1, D)),
  ]

  ego = ego0
  norms = []
  for (wgt, bg, wbt, bb) in params:
    side = _spmm(ego.reshape(2 * N, H), rows2d, cols_p, vals_p, zeros)
    ego, norm = _dense(ego, side[0], side[1], wgt, bg, wbt, bb)
    norms.append(norm)

  all_emb = jnp.concatenate([ego0, norms[0], norms[1]], axis=1)
  return all_emb[:N_U], all_emb[N_U:]


# speedup vs baseline: 1.5858x; 1.0239x over previous
import functools
import jax
import jax.numpy as jnp
from jax import lax
from jax.experimental import pallas as pl
from jax.experimental.pallas import tpu as pltpu
from jax.experimental.pallas import tpu_sc as plsc

N_U = 25000
N_I = 25000
N = N_U + N_I
E = 800000
D = 64
H = D // 2

NUM_CORES = 2
NUM_TILES = 16
BLK = 256
CHUNK = 128
CPB = BLK // CHUNK
EPT_BLKS = 196
EPT = EPT_BLKS * BLK
E_PAD = NUM_TILES * EPT
RPT = 3128
RPT_LAST = N - 15 * RPT


def _spmm_body(tab0_hbm, tab1_hbm, rows_hbm, cols_hbm, vals_hbm, zeros_hbm,
               out0_hbm, out1_hbm,
               rowsb, colsb, valsb, gbuf, accum, isem, gsem, ssem):
  c = lax.axis_index("c")
  t = lax.axis_index("s")
  NB = EPT_BLKS

  def idx_fire(b, p):
    base = pl.multiple_of((t * NB + b) * BLK, BLK)
    roff = pl.multiple_of((t * NB + b) * CPB, CPB)
    pltpu.async_copy(cols_hbm.at[pl.ds(roff, CPB)], colsb.at[p], isem)
    pltpu.async_copy(vals_hbm.at[pl.ds(base, BLK)], valsb.at[p], isem)
    pltpu.async_copy(rows_hbm.at[pl.ds(roff, CPB)], rowsb.at[p], isem)

  def idx_drain(p):
    pltpu.make_async_copy(cols_hbm.at[pl.ds(0, CPB)], colsb.at[p], isem).wait()
    pltpu.make_async_copy(vals_hbm.at[pl.ds(0, BLK)], valsb.at[p], isem).wait()
    pltpu.make_async_copy(rows_hbm.at[pl.ds(0, CPB)], rowsb.at[p], isem).wait()

  def gather_fire(p, g):
    for j in range(CPB):
      @pl.when(c == 0)
      def _():
        pltpu.async_copy(tab0_hbm.at[colsb.at[p, j]],
                         gbuf.at[g, pl.ds(j * CHUNK, CHUNK)], gsem)

      @pl.when(c == 1)
      def _():
        pltpu.async_copy(tab1_hbm.at[colsb.at[p, j]],
                         gbuf.at[g, pl.ds(j * CHUNK, CHUNK)], gsem)

  def gather_drain(p, g):
    for j in range(CPB):
      pltpu.make_async_copy(tab0_hbm.at[colsb.at[p, j]],
                            gbuf.at[g, pl.ds(j * CHUNK, CHUNK)], gsem).wait()

  def scale(p, g):
    def group(i, carry):
      v = valsb[p, pl.ds(i * 16, 16)]
      for e in range(16):
        valv = jnp.broadcast_to(v[e], (16,))
        r = i * 16 + e
        gbuf[g, r, pl.ds(0, 16)] = gbuf[g, r, pl.ds(0, 16)] * valv
        gbuf[g, r, pl.ds(16, 16)] = gbuf[g, r, pl.ds(16, 16)] * valv
      return carry
    lax.fori_loop(0, BLK // 16, group, 0)

  def scatter_fire(p, g):
    for j in range(CPB):
      pltpu.async_copy(gbuf.at[g, pl.ds(j * CHUNK, CHUNK)],
                       accum.at[rowsb.at[p, j]], ssem, add=True)

  def scatter_drain(p, g):
    for j in range(CPB):
      pltpu.make_async_copy(gbuf.at[g, pl.ds(j * CHUNK, CHUNK)],
                            accum.at[rowsb.at[p, j]], ssem).wait()

  off = pl.multiple_of(t * RPT, 8)

  @pl.when(t < NUM_TILES - 1)
  def _():
    pltpu.sync_copy(zeros_hbm, accum.at[pl.ds(off, RPT)])

  @pl.when(t == NUM_TILES - 1)
  def _():
    pltpu.sync_copy(zeros_hbm.at[pl.ds(0, RPT_LAST)],
                    accum.at[pl.ds(off, RPT_LAST)])

  plsc.subcore_barrier()

  idx_fire(0, 0)
  idx_drain(0)
  gather_fire(0, 0)
  idx_fire(1, 1)

  def bump(x, m):
    return jnp.where(x == m - 1, 0, x + 1)

  def block(b, carry):
    (p4, g3, q4, h3, r4, s4, u3) = carry

    gather_drain(p4, g3)

    @pl.when(b + 1 < NB)
    def _():
      idx_drain(q4)
      gather_fire(q4, h3)

      @pl.when(b + 2 < NB)
      def _():
        idx_fire(b + 2, r4)

    scale(p4, g3)

    @pl.when(b >= 1)
    def _():
      scatter_drain(s4, u3)

    scatter_fire(p4, g3)
    return (q4, h3, r4, bump(h3, 3), bump(r4, 4), p4, g3)

  init = (jnp.int32(0), jnp.int32(0), jnp.int32(1), jnp.int32(1),
          jnp.int32(2), jnp.int32(3), jnp.int32(2))
  last = lax.fori_loop(0, NB, block, init)
  scatter_drain(last[5], last[6])
  plsc.subcore_barrier()

  @pl.when(t < NUM_TILES - 1)
  def _():
    sl = pl.ds(off, RPT)

    @pl.when(c == 0)
    def _():
      pltpu.sync_copy(accum.at[sl], out0_hbm.at[sl])

    @pl.when(c == 1)
    def _():
      pltpu.sync_copy(accum.at[sl], out1_hbm.at[sl])

  @pl.when(t == NUM_TILES - 1)
  def _():
    sl = pl.ds(off, RPT_LAST)

    @pl.when(c == 0)
    def _():
      pltpu.sync_copy(accum.at[sl], out0_hbm.at[sl])

    @pl.when(c == 1)
    def _():
      pltpu.sync_copy(accum.at[sl], out1_hbm.at[sl])


_spmm = pl.kernel(
    _spmm_body,
    out_type=[jax.ShapeDtypeStruct((N, H), jnp.float32),
              jax.ShapeDtypeStruct((N, H), jnp.float32)],
    mesh=plsc.VectorSubcoreMesh(core_axis_name="c", subcore_axis_name="s"),
    scratch_types=[
        pltpu.VMEM((4, CPB, CHUNK), jnp.int32),
        pltpu.VMEM((4, CPB, CHUNK), jnp.int32),
        pltpu.VMEM((4, BLK), jnp.float32),
        pltpu.VMEM((3, BLK, H), jnp.float32),
        pltpu.VMEM_SHARED((N, H), jnp.float32),
        pltpu.SemaphoreType.DMA,
        pltpu.SemaphoreType.DMA,
        pltpu.SemaphoreType.DMA,
    ],
    compiler_params=pltpu.CompilerParams(use_tc_tiling_on_sc=False),
)


def _dense_body(eh0_ref, eh1_ref, sh0_ref, sh1_ref, wg_ref, bg_ref, wb_ref,
                bb_ref, oh0_ref, oh1_ref, norm_ref):
  s = jnp.concatenate([sh0_ref[...], sh1_ref[...]], axis=1)
  ego = jnp.concatenate([eh0_ref[...], eh1_ref[...]], axis=1)
  x = jnp.dot(s, wg_ref[...], preferred_element_type=jnp.float32) + bg_ref[...]
  sum_emb = jnp.where(x > 0, x, 0.01 * x)
  y = jnp.dot(ego * s, wb_ref[...], preferred_element_type=jnp.float32) + bb_ref[...]
  bi = jnp.where(y > 0, y, 0.01 * y)
  e2 = sum_emb + bi
  nrm = jnp.sqrt(jnp.sum(e2 * e2, axis=1, keepdims=True))
  oh0_ref[...] = e2[:, :H]
  oh1_ref[...] = e2[:, H:]
  norm_ref[...] = e2 / jnp.maximum(nrm, 1e-12)


_BN = 2000


def _dense(eh0, eh1, sh0, sh1, wgt, bg, wbt, bb):
  return pl.pallas_call(
      _dense_body,
      grid=(N // _BN,),
      in_specs=[
          pl.BlockSpec((_BN, H), lambda i: (i, 0)),
          pl.BlockSpec((_BN, H), lambda i: (i, 0)),
          pl.BlockSpec((_BN, H), lambda i: (i, 0)),
          pl.BlockSpec((_BN, H), lambda i: (i, 0)),
          pl.BlockSpec((D, D), lambda i: (0, 0)),
          pl.BlockSpec((1, D), lambda i: (0, 0)),
          pl.BlockSpec((D, D), lambda i: (0, 0)),
          pl.BlockSpec((1, D), lambda i: (0, 0)),
      ],
      out_specs=[
          pl.BlockSpec((_BN, H), lambda i: (i, 0)),
          pl.BlockSpec((_BN, H), lambda i: (i, 0)),
          pl.BlockSpec((_BN, D), lambda i: (i, 0)),
      ],
      out_shape=[
          jax.ShapeDtypeStruct((N, H), jnp.float32),
          jax.ShapeDtypeStruct((N, H), jnp.float32),
          jax.ShapeDtypeStruct((N, D), jnp.float32),
      ],
  )(eh0, eh1, sh0, sh1, wgt, bg, wbt, bb)


def kernel(adj_rows, adj_cols, adj_vals, user_emb, item_emb,
           W_gc0, b_gc0, W_bi0, b_bi0, W_gc1, b_gc1, W_bi1, b_bi1):
  rows = adj_rows.astype(jnp.int32)
  cols = adj_cols.astype(jnp.int32)
  vals = adj_vals.astype(jnp.float32)
  pad = E_PAD - E
  rows_p = jnp.concatenate([rows, jnp.zeros((pad,), jnp.int32)])
  cols_p = jnp.concatenate([cols, jnp.zeros((pad,), jnp.int32)])
  vals_p = jnp.concatenate([vals, jnp.zeros((pad,), jnp.float32)])
  rows2d = rows_p.reshape(E_PAD // CHUNK, CHUNK)
  cols2d = cols_p.reshape(E_PAD // CHUNK, CHUNK)
  zeros = jnp.zeros((RPT, H), jnp.float32)

  ego0 = jnp.concatenate([user_emb, item_emb], axis=0)
  eh0 = ego0[:, :H]
  eh1 = ego0[:, H:]
  params = [
      (W_gc0.T, b_gc0.reshape(1, D), W_bi0.T, b_bi0.reshape(1, D)),
      (W_gc1.T, b_gc1.reshape(1, D), W_bi1.T, b_bi1.reshape(1, D)),
  ]

  norms = []
  for (wgt, bg, wbt, bb) in params:
    sh0, sh1 = _spmm(eh0, eh1, rows2d, cols2d, vals_p, zeros)
    eh0, eh1, norm = _dense(eh0, eh1, sh0, sh1, wgt, bg, wbt, bb)
    norms.append(norm)

  all_emb = jnp.concatenate([ego0, norms[0], norms[1]], axis=1)
  return all_emb[:N_U], all_emb[N_U:]


# speedup vs baseline: 1.6001x; 1.0091x over previous
import functools
import jax
import jax.numpy as jnp
from jax import lax
from jax.experimental import pallas as pl
from jax.experimental.pallas import tpu as pltpu
from jax.experimental.pallas import tpu_sc as plsc

N_U = 25000
N_I = 25000
N = N_U + N_I
E = 800000
D = 64
H = D // 2

NUM_CORES = 2
NUM_TILES = 16
BLK = 256
CHUNK = 128
CPB = BLK // CHUNK
EPT_BLKS = 196
EPT = EPT_BLKS * BLK
E_PAD = NUM_TILES * EPT
RPT = 3128
RPT_LAST = N - 15 * RPT


def _spmm_body(tab0_hbm, tab1_hbm, rows_hbm, cols_hbm, vals_hbm, zeros_hbm,
               out0_hbm, out1_hbm,
               rowsb, colsb, valsb, gbuf, accum, isem, gsem, ssem):
  c = lax.axis_index("c")
  t = lax.axis_index("s")
  NB = EPT_BLKS

  def idx_fire(b, p):
    base = pl.multiple_of((t * NB + b) * BLK, BLK)
    roff = pl.multiple_of((t * NB + b) * CPB, CPB)
    pltpu.async_copy(cols_hbm.at[pl.ds(roff, CPB)], colsb.at[p], isem)
    pltpu.async_copy(vals_hbm.at[pl.ds(base, BLK)], valsb.at[p], isem)
    pltpu.async_copy(rows_hbm.at[pl.ds(roff, CPB)], rowsb.at[p], isem)

  def idx_drain(p):
    pltpu.make_async_copy(cols_hbm.at[pl.ds(0, CPB)], colsb.at[p], isem).wait()
    pltpu.make_async_copy(vals_hbm.at[pl.ds(0, BLK)], valsb.at[p], isem).wait()
    pltpu.make_async_copy(rows_hbm.at[pl.ds(0, CPB)], rowsb.at[p], isem).wait()

  def gather_fire(p, g):
    for j in range(CPB):
      @pl.when(c == 0)
      def _():
        pltpu.async_copy(tab0_hbm.at[colsb.at[p, j]],
                         gbuf.at[g, pl.ds(j * CHUNK, CHUNK)], gsem)

      @pl.when(c == 1)
      def _():
        pltpu.async_copy(tab1_hbm.at[colsb.at[p, j]],
                         gbuf.at[g, pl.ds(j * CHUNK, CHUNK)], gsem)

  def gather_drain(p, g):
    for j in range(CPB):
      pltpu.make_async_copy(tab0_hbm.at[colsb.at[p, j]],
                            gbuf.at[g, pl.ds(j * CHUNK, CHUNK)], gsem).wait()

  def scale(p, g):
    def group(i, carry):
      v = valsb[p, pl.ds(i * 16, 16)]
      for e in range(16):
        valv = jnp.broadcast_to(v[e], (16,))
        r = i * 16 + e
        gbuf[g, r, pl.ds(0, 16)] = gbuf[g, r, pl.ds(0, 16)] * valv
        gbuf[g, r, pl.ds(16, 16)] = gbuf[g, r, pl.ds(16, 16)] * valv
      return carry
    lax.fori_loop(0, BLK // 16, group, 0)

  def scatter_fire(p, g):
    for j in range(CPB):
      pltpu.async_copy(gbuf.at[g, pl.ds(j * CHUNK, CHUNK)],
                       accum.at[rowsb.at[p, j]], ssem, add=True)

  def scatter_drain(p, g):
    for j in range(CPB):
      pltpu.make_async_copy(gbuf.at[g, pl.ds(j * CHUNK, CHUNK)],
                            accum.at[rowsb.at[p, j]], ssem).wait()

  off = pl.multiple_of(t * RPT, 8)

  @pl.when(t < NUM_TILES - 1)
  def _():
    pltpu.sync_copy(zeros_hbm, accum.at[pl.ds(off, RPT)])

  @pl.when(t == NUM_TILES - 1)
  def _():
    pltpu.sync_copy(zeros_hbm.at[pl.ds(0, RPT_LAST)],
                    accum.at[pl.ds(off, RPT_LAST)])

  plsc.subcore_barrier()

  idx_fire(0, 0)
  idx_drain(0)
  gather_fire(0, 0)
  idx_fire(1, 1)

  def bump(x, m):
    return jnp.where(x == m - 1, 0, x + 1)

  def block(b, carry):
    (p4, g3, q4, h3, r4, s4, u3) = carry

    gather_drain(p4, g3)

    @pl.when(b + 1 < NB)
    def _():
      idx_drain(q4)
      gather_fire(q4, h3)

      @pl.when(b + 2 < NB)
      def _():
        idx_fire(b + 2, r4)

    scale(p4, g3)

    @pl.when(b >= 1)
    def _():
      scatter_drain(s4, u3)

    scatter_fire(p4, g3)
    return (q4, h3, r4, bump(h3, 3), bump(r4, 4), p4, g3)

  init = (jnp.int32(0), jnp.int32(0), jnp.int32(1), jnp.int32(1),
          jnp.int32(2), jnp.int32(3), jnp.int32(2))
  last = lax.fori_loop(0, NB, block, init)
  scatter_drain(last[5], last[6])
  plsc.subcore_barrier()

  @pl.when(t < NUM_TILES - 1)
  def _():
    sl = pl.ds(off, RPT)

    @pl.when(c == 0)
    def _():
      pltpu.sync_copy(accum.at[sl], out0_hbm.at[sl])

    @pl.when(c == 1)
    def _():
      pltpu.sync_copy(accum.at[sl], out1_hbm.at[sl])

  @pl.when(t == NUM_TILES - 1)
  def _():
    sl = pl.ds(off, RPT_LAST)

    @pl.when(c == 0)
    def _():
      pltpu.sync_copy(accum.at[sl], out0_hbm.at[sl])

    @pl.when(c == 1)
    def _():
      pltpu.sync_copy(accum.at[sl], out1_hbm.at[sl])


_spmm = pl.kernel(
    _spmm_body,
    out_type=[jax.ShapeDtypeStruct((N, H), jnp.float32),
              jax.ShapeDtypeStruct((N, H), jnp.float32)],
    mesh=plsc.VectorSubcoreMesh(core_axis_name="c", subcore_axis_name="s"),
    scratch_types=[
        pltpu.VMEM((4, CPB, CHUNK), jnp.int32),
        pltpu.VMEM((4, CPB, CHUNK), jnp.int32),
        pltpu.VMEM((4, BLK), jnp.float32),
        pltpu.VMEM((3, BLK, H), jnp.float32),
        pltpu.VMEM_SHARED((N, H), jnp.float32),
        pltpu.SemaphoreType.DMA,
        pltpu.SemaphoreType.DMA,
        pltpu.SemaphoreType.DMA,
    ],
    compiler_params=pltpu.CompilerParams(use_tc_tiling_on_sc=False),
)


def _dense_body(eh0_ref, eh1_ref, sh0_ref, sh1_ref, wg_ref, bg_ref, wb_ref,
                bb_ref, oh0_ref, oh1_ref, norm_ref):
  s = jnp.concatenate([sh0_ref[...], sh1_ref[...]], axis=1)
  ego = jnp.concatenate([eh0_ref[...], eh1_ref[...]], axis=1)
  x = jnp.dot(s, wg_ref[...], preferred_element_type=jnp.float32) + bg_ref[...]
  sum_emb = jnp.where(x > 0, x, 0.01 * x)
  y = jnp.dot(ego * s, wb_ref[...], preferred_element_type=jnp.float32) + bb_ref[...]
  bi = jnp.where(y > 0, y, 0.01 * y)
  e2 = sum_emb + bi
  nrm = jnp.sqrt(jnp.sum(e2 * e2, axis=1, keepdims=True))
  oh0_ref[...] = e2[:, :H]
  oh1_ref[...] = e2[:, H:]
  norm_ref[...] = e2 / jnp.maximum(nrm, 1e-12)


_BN = 5000


def _dense(eh0, eh1, sh0, sh1, wgt, bg, wbt, bb):
  return pl.pallas_call(
      _dense_body,
      grid=(N // _BN,),
      in_specs=[
          pl.BlockSpec((_BN, H), lambda i: (i, 0)),
          pl.BlockSpec((_BN, H), lambda i: (i, 0)),
          pl.BlockSpec((_BN, H), lambda i: (i, 0)),
          pl.BlockSpec((_BN, H), lambda i: (i, 0)),
          pl.BlockSpec((D, D), lambda i: (0, 0)),
          pl.BlockSpec((1, D), lambda i: (0, 0)),
          pl.BlockSpec((D, D), lambda i: (0, 0)),
          pl.BlockSpec((1, D), lambda i: (0, 0)),
      ],
      out_specs=[
          pl.BlockSpec((_BN, H), lambda i: (i, 0)),
          pl.BlockSpec((_BN, H), lambda i: (i, 0)),
          pl.BlockSpec((_BN, D), lambda i: (i, 0)),
      ],
      out_shape=[
          jax.ShapeDtypeStruct((N, H), jnp.float32),
          jax.ShapeDtypeStruct((N, H), jnp.float32),
          jax.ShapeDtypeStruct((N, D), jnp.float32),
      ],
  )(eh0, eh1, sh0, sh1, wgt, bg, wbt, bb)


def kernel(adj_rows, adj_cols, adj_vals, user_emb, item_emb,
           W_gc0, b_gc0, W_bi0, b_bi0, W_gc1, b_gc1, W_bi1, b_bi1):
  rows = adj_rows.astype(jnp.int32)
  cols = adj_cols.astype(jnp.int32)
  vals = adj_vals.astype(jnp.float32)
  pad = E_PAD - E
  rows_p = jnp.concatenate([rows, jnp.zeros((pad,), jnp.int32)])
  cols_p = jnp.concatenate([cols, jnp.zeros((pad,), jnp.int32)])
  vals_p = jnp.concatenate([vals, jnp.zeros((pad,), jnp.float32)])
  rows2d = rows_p.reshape(E_PAD // CHUNK, CHUNK)
  cols2d = cols_p.reshape(E_PAD // CHUNK, CHUNK)
  zeros = jnp.zeros((RPT, H), jnp.float32)

  ego0 = jnp.concatenate([user_emb, item_emb], axis=0)
  eh0 = ego0[:, :H]
  eh1 = ego0[:, H:]
  params = [
      (W_gc0.T, b_gc0.reshape(1, D), W_bi0.T, b_bi0.reshape(1, D)),
      (W_gc1.T, b_gc1.reshape(1, D), W_bi1.T, b_bi1.reshape(1, D)),
  ]

  norms = []
  for (wgt, bg, wbt, bb) in params:
    sh0, sh1 = _spmm(eh0, eh1, rows2d, cols2d, vals_p, zeros)
    eh0, eh1, norm = _dense(eh0, eh1, sh0, sh1, wgt, bg, wbt, bb)
    norms.append(norm)

  all_emb = jnp.concatenate([ego0, norms[0], norms[1]], axis=1)
  return all_emb[:N_U], all_emb[N_U:]


# speedup vs baseline: 1.6024x; 1.0014x over previous
import jax
import jax.numpy as jnp
from jax import lax
from jax.experimental import pallas as pl
from jax.experimental.pallas import tpu as pltpu
from jax.experimental.pallas import tpu_sc as plsc

N_U = 25000
N_I = 25000
N = N_U + N_I
E = 800000
D = 64
H = D // 2

NUM_CORES = 2
NUM_TILES = 16
BLK = 256
CHUNK = 128
CPB = BLK // CHUNK
EPT_BLKS = 196
EPT = EPT_BLKS * BLK
E_PAD = NUM_TILES * EPT
RPT = 3128
RPT_LAST = N - 15 * RPT


def _spmm_body(tab0_hbm, tab1_hbm, rows_hbm, cols_hbm, vals_hbm, zeros_hbm,
               out0_hbm, out1_hbm,
               rowsb, colsb, valsb, gbuf, accum, isem, gsem, ssem):
  c = lax.axis_index("c")
  t = lax.axis_index("s")
  NB = EPT_BLKS

  def idx_fire(b, p):
    base = pl.multiple_of((t * NB + b) * BLK, BLK)
    roff = pl.multiple_of((t * NB + b) * CPB, CPB)
    pltpu.async_copy(cols_hbm.at[pl.ds(roff, CPB)], colsb.at[p], isem)
    pltpu.async_copy(vals_hbm.at[pl.ds(base, BLK)], valsb.at[p], isem)
    pltpu.async_copy(rows_hbm.at[pl.ds(roff, CPB)], rowsb.at[p], isem)

  def idx_drain(p):
    pltpu.make_async_copy(cols_hbm.at[pl.ds(0, CPB)], colsb.at[p], isem).wait()
    pltpu.make_async_copy(vals_hbm.at[pl.ds(0, BLK)], valsb.at[p], isem).wait()
    pltpu.make_async_copy(rows_hbm.at[pl.ds(0, CPB)], rowsb.at[p], isem).wait()

  def gather_fire(p, g):
    for j in range(CPB):
      @pl.when(c == 0)
      def _():
        pltpu.async_copy(tab0_hbm.at[colsb.at[p, j]],
                         gbuf.at[g, pl.ds(j * CHUNK, CHUNK)], gsem)

      @pl.when(c == 1)
      def _():
        pltpu.async_copy(tab1_hbm.at[colsb.at[p, j]],
                         gbuf.at[g, pl.ds(j * CHUNK, CHUNK)], gsem)

  def gather_drain(p, g):
    for j in range(CPB):
      pltpu.make_async_copy(tab0_hbm.at[colsb.at[p, j]],
                            gbuf.at[g, pl.ds(j * CHUNK, CHUNK)], gsem).wait()

  def scale(p, g):
    def group(i, carry):
      v = valsb[p, pl.ds(i * 16, 16)]
      for e in range(16):
        valv = jnp.broadcast_to(v[e], (16,))
        r = i * 16 + e
        gbuf[g, r, pl.ds(0, 16)] = gbuf[g, r, pl.ds(0, 16)] * valv
        gbuf[g, r, pl.ds(16, 16)] = gbuf[g, r, pl.ds(16, 16)] * valv
      return carry
    lax.fori_loop(0, BLK // 16, group, 0)

  def scatter_fire(p, g):
    for j in range(CPB):
      pltpu.async_copy(gbuf.at[g, pl.ds(j * CHUNK, CHUNK)],
                       accum.at[rowsb.at[p, j]], ssem, add=True)

  def scatter_drain(p, g):
    for j in range(CPB):
      pltpu.make_async_copy(gbuf.at[g, pl.ds(j * CHUNK, CHUNK)],
                            accum.at[rowsb.at[p, j]], ssem).wait()

  off = pl.multiple_of(t * RPT, 8)

  @pl.when(t < NUM_TILES - 1)
  def _():
    pltpu.sync_copy(zeros_hbm, accum.at[pl.ds(off, RPT)])

  @pl.when(t == NUM_TILES - 1)
  def _():
    pltpu.sync_copy(zeros_hbm.at[pl.ds(0, RPT_LAST)],
                    accum.at[pl.ds(off, RPT_LAST)])

  plsc.subcore_barrier()

  idx_fire(0, 0)
  idx_drain(0)
  gather_fire(0, 0)
  idx_fire(1, 1)

  def bump(x, m):
    return jnp.where(x == m - 1, 0, x + 1)

  def block(b, carry):
    (p4, g3, q4, h3, r4, s4, u3) = carry

    gather_drain(p4, g3)

    @pl.when(b + 1 < NB)
    def _():
      idx_drain(q4)
      gather_fire(q4, h3)

      @pl.when(b + 2 < NB)
      def _():
        idx_fire(b + 2, r4)

    scale(p4, g3)

    @pl.when(b >= 1)
    def _():
      scatter_drain(s4, u3)

    scatter_fire(p4, g3)
    return (q4, h3, r4, bump(h3, 3), bump(r4, 4), p4, g3)

  init = (jnp.int32(0), jnp.int32(0), jnp.int32(1), jnp.int32(1),
          jnp.int32(2), jnp.int32(3), jnp.int32(2))
  last = lax.fori_loop(0, NB, block, init)
  scatter_drain(last[5], last[6])
  plsc.subcore_barrier()

  @pl.when(t < NUM_TILES - 1)
  def _():
    sl = pl.ds(off, RPT)

    @pl.when(c == 0)
    def _():
      pltpu.sync_copy(accum.at[sl], out0_hbm.at[sl])

    @pl.when(c == 1)
    def _():
      pltpu.sync_copy(accum.at[sl], out1_hbm.at[sl])

  @pl.when(t == NUM_TILES - 1)
  def _():
    sl = pl.ds(off, RPT_LAST)

    @pl.when(c == 0)
    def _():
      pltpu.sync_copy(accum.at[sl], out0_hbm.at[sl])

    @pl.when(c == 1)
    def _():
      pltpu.sync_copy(accum.at[sl], out1_hbm.at[sl])


_spmm = pl.kernel(
    _spmm_body,
    out_type=[jax.ShapeDtypeStruct((N, H), jnp.float32),
              jax.ShapeDtypeStruct((N, H), jnp.float32)],
    mesh=plsc.VectorSubcoreMesh(core_axis_name="c", subcore_axis_name="s"),
    scratch_types=[
        pltpu.VMEM((4, CPB, CHUNK), jnp.int32),
        pltpu.VMEM((4, CPB, CHUNK), jnp.int32),
        pltpu.VMEM((4, BLK), jnp.float32),
        pltpu.VMEM((3, BLK, H), jnp.float32),
        pltpu.VMEM_SHARED((N, H), jnp.float32),
        pltpu.SemaphoreType.DMA,
        pltpu.SemaphoreType.DMA,
        pltpu.SemaphoreType.DMA,
    ],
    compiler_params=pltpu.CompilerParams(use_tc_tiling_on_sc=False),
)


def _dense_body(eh0_ref, eh1_ref, sh0_ref, sh1_ref, wg_ref, bg_ref, wb_ref,
                bb_ref, oh0_ref, oh1_ref, norm_ref):
  s = jnp.concatenate([sh0_ref[...], sh1_ref[...]], axis=1)
  ego = jnp.concatenate([eh0_ref[...], eh1_ref[...]], axis=1)
  x = jnp.dot(s, wg_ref[...], preferred_element_type=jnp.float32) + bg_ref[...]
  sum_emb = jnp.where(x > 0, x, 0.01 * x)
  y = jnp.dot(ego * s, wb_ref[...], preferred_element_type=jnp.float32) + bb_ref[...]
  bi = jnp.where(y > 0, y, 0.01 * y)
  e2 = sum_emb + bi
  nrm = jnp.sqrt(jnp.sum(e2 * e2, axis=1, keepdims=True))
  oh0_ref[...] = e2[:, :H]
  oh1_ref[...] = e2[:, H:]
  norm_ref[...] = e2 / jnp.maximum(nrm, 1e-12)


_BN = 5000


def _dense(eh0, eh1, sh0, sh1, wgt, bg, wbt, bb):
  return pl.pallas_call(
      _dense_body,
      grid=(N // _BN,),
      in_specs=[
          pl.BlockSpec((_BN, H), lambda i: (i, 0)),
          pl.BlockSpec((_BN, H), lambda i: (i, 0)),
          pl.BlockSpec((_BN, H), lambda i: (i, 0)),
          pl.BlockSpec((_BN, H), lambda i: (i, 0)),
          pl.BlockSpec((D, D), lambda i: (0, 0)),
          pl.BlockSpec((1, D), lambda i: (0, 0)),
          pl.BlockSpec((D, D), lambda i: (0, 0)),
          pl.BlockSpec((1, D), lambda i: (0, 0)),
      ],
      out_specs=[
          pl.BlockSpec((_BN, H), lambda i: (i, 0)),
          pl.BlockSpec((_BN, H), lambda i: (i, 0)),
          pl.BlockSpec((_BN, D), lambda i: (i, 0)),
      ],
      out_shape=[
          jax.ShapeDtypeStruct((N, H), jnp.float32),
          jax.ShapeDtypeStruct((N, H), jnp.float32),
          jax.ShapeDtypeStruct((N, D), jnp.float32),
      ],
  )(eh0, eh1, sh0, sh1, wgt, bg, wbt, bb)


def kernel(adj_rows, adj_cols, adj_vals, user_emb, item_emb,
           W_gc0, b_gc0, W_bi0, b_bi0, W_gc1, b_gc1, W_bi1, b_bi1):
  rows = adj_rows.astype(jnp.int32)
  cols = adj_cols.astype(jnp.int32)
  vals = adj_vals.astype(jnp.float32)
  pad = E_PAD - E
  rows_p = jnp.concatenate([rows, jnp.zeros((pad,), jnp.int32)])
  cols_p = jnp.concatenate([cols, jnp.zeros((pad,), jnp.int32)])
  vals_p = jnp.concatenate([vals, jnp.zeros((pad,), jnp.float32)])
  rows2d = rows_p.reshape(E_PAD // CHUNK, CHUNK)
  cols2d = cols_p.reshape(E_PAD // CHUNK, CHUNK)
  zeros = jnp.zeros((RPT, H), jnp.float32)

  ego0 = jnp.concatenate([user_emb, item_emb], axis=0)
  eh0 = ego0[:, :H]
  eh1 = ego0[:, H:]
  params = [
      (W_gc0.T, b_gc0.reshape(1, D), W_bi0.T, b_bi0.reshape(1, D)),
      (W_gc1.T, b_gc1.reshape(1, D), W_bi1.T, b_bi1.reshape(1, D)),
  ]

  norms = []
  for (wgt, bg, wbt, bb) in params:
    sh0, sh1 = _spmm(eh0, eh1, rows2d, cols2d, vals_p, zeros)
    eh0, eh1, norm = _dense(eh0, eh1, sh0, sh1, wgt, bg, wbt, bb)
    norms.append(norm)

  all_emb = jnp.concatenate([ego0, norms[0], norms[1]], axis=1)
  return all_emb[:N_U], all_emb[N_U:]
